# baseline (speedup 1.0000x reference)
import functools

import jax
import jax.numpy as jnp
from jax import lax
from jax.experimental import pallas as pl
from jax.experimental.pallas import tpu as pltpu
from jax.experimental.pallas import tpu_sc as plsc

VOCAB = 100000
D = 768
B = 4
S = 2048
NC = 2
NS = 16
NW = NC * NS
RPW = S // NW
PPC = 8
NCHUNK = RPW // PPC
NBUF = 3
LANES = 16
HALF = D // 2
VECS_PER_HALF = HALF // LANES


def _body(ids_hbm, tok_hbm, pos_hbm, out_hbm,
          idx_v, pos0, pos1, pos2, tok0, tok1, tok2,
          sem_ids, sp0, sp1, sp2, sg0, sg1, sg2, sw0, sw1, sw2):
    wid = lax.axis_index("s") * NC + lax.axis_index("c")
    base = wid * RPW

    poss = (pos0, pos1, pos2)
    toks = (tok0, tok1, tok2)
    psems = (sp0, sp1, sp2)
    gsems = (sg0, sg1, sg2)
    wsems = (sw0, sw1, sw2)

    ih = []
    for c in range(NCHUNK):
        for b in range(B):
            h = pltpu.make_async_copy(
                ids_hbm.at[b, pl.ds(base + c * PPC, PPC)],
                idx_v.at[pl.ds(c * PPC * B + b * PPC, PPC)], sem_ids)
            h.start()
            ih.append(h)

    def start_pos(c):
        k = c % NBUF
        h = pltpu.make_async_copy(
            pos_hbm.at[pl.ds(base + c * PPC, PPC)], poss[k], psems[k])
        h.start()
        return h

    ph = [start_pos(0), start_pos(1), start_pos(2)]
    for h in ih:
        h.wait()

    gh = [None] * NBUF

    def start_gather(c):
        k = c % NBUF
        gh[k] = pltpu.make_async_copy(
            tok_hbm.at[idx_v.at[pl.ds(c * PPC * B, PPC * B)]],
            toks[k], gsems[k])
        gh[k].start()

    for c in range(NBUF):
        start_gather(c)

    wh = [None] * NBUF
    for c in range(NCHUNK):
        k = c % NBUF
        gh[k].wait()
        ph[k].wait()
        tok_v = toks[k]
        pos_v = poss[k]

        def pos_body(p, c2, tok_v=tok_v, pos_v=pos_v):
            pv = pos_v.at[p]
            for hh in range(2):
                pvec = [pv[pl.ds(hh * HALF + j * LANES, LANES)]
                        for j in range(VECS_PER_HALF)]
                for b in range(B):
                    tv = tok_v.at[b * PPC + p]
                    for j in range(VECS_PER_HALF):
                        sl = pl.ds(hh * HALF + j * LANES, LANES)
                        tv[sl] = tv[sl] + pvec[j]
            return c2

        lax.fori_loop(0, PPC, pos_body, 0)

        wh[k] = []
        for b in range(B):
            w = pltpu.make_async_copy(
                tok_v.at[pl.ds(b * PPC, PPC)],
                out_hbm.at[b, pl.ds(base + c * PPC, PPC)], wsems[k])
            w.start()
            wh[k].append(w)

        if c + NBUF < NCHUNK:
            for w in wh[k]:
                w.wait()
            wh[k] = None
            ph[k] = start_pos(c + NBUF)
            start_gather(c + NBUF)

    for k in range(NBUF):
        if wh[k] is not None:
            for w in wh[k]:
                w.wait()


@functools.partial(jax.jit, static_argnames=())
def _embed(input_ids, token_table, position_table):
    mesh = plsc.VectorSubcoreMesh(core_axis_name="c", subcore_axis_name="s")
    run = pl.kernel(
        _body,
        out_type=jax.ShapeDtypeStruct((B, S, D), jnp.float32),
        mesh=mesh,
        scratch_types=[
            pltpu.VMEM((B * RPW,), jnp.int32),
            pltpu.VMEM((PPC, D), jnp.float32),
            pltpu.VMEM((PPC, D), jnp.float32),
            pltpu.VMEM((PPC, D), jnp.float32),
            pltpu.VMEM((PPC * B, D), jnp.float32),
            pltpu.VMEM((PPC * B, D), jnp.float32),
            pltpu.VMEM((PPC * B, D), jnp.float32),
            pltpu.SemaphoreType.DMA,
            pltpu.SemaphoreType.DMA,
            pltpu.SemaphoreType.DMA,
            pltpu.SemaphoreType.DMA,
            pltpu.SemaphoreType.DMA,
            pltpu.SemaphoreType.DMA,
            pltpu.SemaphoreType.DMA,
            pltpu.SemaphoreType.DMA,
            pltpu.SemaphoreType.DMA,
            pltpu.SemaphoreType.DMA,
        ],
    )
    return run(input_ids, token_table, position_table)


def kernel(input_ids, token_table, position_table):
    return _embed(input_ids.astype(jnp.int32), token_table, position_table)


# speedup vs baseline: 1.0158x; 1.0158x over previous
import functools

import jax
import jax.numpy as jnp
from jax import lax
from jax.experimental import pallas as pl
from jax.experimental.pallas import tpu as pltpu
from jax.experimental.pallas import tpu_sc as plsc

VOCAB = 100000
D = 768
B = 4
S = 2048
NC = 2
NS = 16
NW = NC * NS
RPW = S // NW
PPC = 16
NCHUNK = RPW // PPC
LANES = 16
HALF = D // 2
VECS_PER_HALF = HALF // LANES


def _body(ids_hbm, tok_hbm, pos_hbm, out_hbm,
          idx_v, pos0, pos1, tok0, tok1,
          sem_ids, sp0, sp1, sg0, sg1, sw0, sw1):
    wid = lax.axis_index("s") * NC + lax.axis_index("c")
    base = wid * RPW

    poss = (pos0, pos1)
    toks = (tok0, tok1)
    psems = (sp0, sp1)
    gsems = (sg0, sg1)
    wsems = (sw0, sw1)

    ih = []
    for c in range(NCHUNK):
        for b in range(B):
            h = pltpu.make_async_copy(
                ids_hbm.at[b, pl.ds(base + c * PPC, PPC)],
                idx_v.at[pl.ds(c * PPC * B + b * PPC, PPC)], sem_ids)
            h.start()
            ih.append(h)

    def start_pos(c):
        k = c % 2
        h = pltpu.make_async_copy(
            pos_hbm.at[pl.ds(base + c * PPC, PPC)], poss[k], psems[k])
        h.start()
        return h

    ph = [start_pos(0), start_pos(1)]
    for h in ih:
        h.wait()

    gh = [None, None]

    def start_gather(c):
        k = c % 2
        gh[k] = pltpu.make_async_copy(
            tok_hbm.at[idx_v.at[pl.ds(c * PPC * B, PPC * B)]],
            toks[k], gsems[k])
        gh[k].start()

    start_gather(0)
    start_gather(1)

    wh = [None, None]
    for c in range(NCHUNK):
        k = c % 2
        gh[k].wait()
        ph[k].wait()
        tok_v = toks[k]
        pos_v = poss[k]

        def pos_body(p, c2, tok_v=tok_v, pos_v=pos_v):
            pv = pos_v.at[p]
            for hh in range(2):
                pvec = [pv[pl.ds(hh * HALF + j * LANES, LANES)]
                        for j in range(VECS_PER_HALF)]
                for b in range(B):
                    tv = tok_v.at[b * PPC + p]
                    for j in range(VECS_PER_HALF):
                        sl = pl.ds(hh * HALF + j * LANES, LANES)
                        tv[sl] = tv[sl] + pvec[j]
            return c2

        lax.fori_loop(0, PPC, pos_body, 0)

        wh[k] = []
        for b in range(B):
            w = pltpu.make_async_copy(
                tok_v.at[pl.ds(b * PPC, PPC)],
                out_hbm.at[b, pl.ds(base + c * PPC, PPC)], wsems[k])
            w.start()
            wh[k].append(w)

        if c + 2 < NCHUNK:
            for w in wh[k]:
                w.wait()
            wh[k] = None
            ph[k] = start_pos(c + 2)
            start_gather(c + 2)

    for k in range(2):
        if wh[k] is not None:
            for w in wh[k]:
                w.wait()


@functools.partial(jax.jit, static_argnames=())
def _embed(input_ids, token_table, position_table):
    mesh = plsc.VectorSubcoreMesh(core_axis_name="c", subcore_axis_name="s")
    run = pl.kernel(
        _body,
        out_type=jax.ShapeDtypeStruct((B, S, D), jnp.float32),
        mesh=mesh,
        scratch_types=[
            pltpu.VMEM((B * RPW,), jnp.int32),
            pltpu.VMEM((PPC, D), jnp.float32),
            pltpu.VMEM((PPC, D), jnp.float32),
            pltpu.VMEM((PPC * B, D), jnp.float32),
            pltpu.VMEM((PPC * B, D), jnp.float32),
            pltpu.SemaphoreType.DMA,
            pltpu.SemaphoreType.DMA,
            pltpu.SemaphoreType.DMA,
            pltpu.SemaphoreType.DMA,
            pltpu.SemaphoreType.DMA,
            pltpu.SemaphoreType.DMA,
            pltpu.SemaphoreType.DMA,
        ],
    )
    return run(input_ids, token_table, position_table)


def kernel(input_ids, token_table, position_table):
    return _embed(input_ids.astype(jnp.int32), token_table, position_table)
